# Initial kernel scaffold; baseline (speedup 1.0000x reference)
import jax
import jax.numpy as jnp
from jax.experimental import pallas as pl


def kernel(inputs, edge_index, W_emb, W0, al0, ar0, W1, al1, ar1, W2, al2, ar2, O0w, O0b, O1w, O1b, O2w, O2b):
    raise NotImplementedError("write your pallas kernel here")


# speedup vs baseline: 60.5573x; 60.5573x over previous
import functools

import jax
import jax.numpy as jnp
from jax import lax
from jax.experimental import pallas as pl
from jax.experimental.pallas import tpu as pltpu
from jax.experimental.pallas import tpu_sc as plsc

N = 10000
E = 320000
IN_DIM = 128
D = 64
NEG = 0.2

NC = 2
NS = 16
L = 16
NW = NC * NS
EPW = E // NW
CH = 80
NCHUNK = EPW // CH
RPT0 = 624
RPT_LAST = N - (NS - 1) * RPT0
CPR = 208
RB = 2000


def _tc_embed_body(x_ref, wemb_ref, w_ref, a_ref, wh_ref, el_ref, er_ref):
    h = jnp.tanh(x_ref[...] @ wemb_ref[...])
    wh = h @ w_ref[...]
    wh_ref[...] = wh
    ee = wh @ a_ref[...]
    el_ref[...] = ee[:, 0:1]
    er_ref[...] = ee[:, 1:2]


def _tc_embed(x, wemb, w, a):
    return pl.pallas_call(
        _tc_embed_body,
        grid=(N // RB,),
        in_specs=[
            pl.BlockSpec((RB, IN_DIM), lambda i: (i, 0)),
            pl.BlockSpec((IN_DIM, D), lambda i: (0, 0)),
            pl.BlockSpec((D, D), lambda i: (0, 0)),
            pl.BlockSpec((D, 2), lambda i: (0, 0)),
        ],
        out_specs=[
            pl.BlockSpec((RB, D), lambda i: (i, 0)),
            pl.BlockSpec((RB, 1), lambda i: (i, 0)),
            pl.BlockSpec((RB, 1), lambda i: (i, 0)),
        ],
        out_shape=[
            jax.ShapeDtypeStruct((N, D), jnp.float32),
            jax.ShapeDtypeStruct((N, 1), jnp.float32),
            jax.ShapeDtypeStruct((N, 1), jnp.float32),
        ],
    )(x, wemb, w, a)


def _norm_elu(p_ref, sp_ref):
    s = jnp.sum(sp_ref[...], axis=(0, 1))
    x = (p_ref[0] + p_ref[1]) * (1.0 / (s + 1e-9))[:, None]
    return jnp.where(x > 0, x, jnp.exp(x) - 1.0)


def _tc_next_body(p_ref, sp_ref, w_ref, a_ref, g_ref, wh_ref, el_ref, er_ref):
    g = _norm_elu(p_ref, sp_ref)
    g_ref[...] = g
    wh = g @ w_ref[...]
    wh_ref[...] = wh
    ee = wh @ a_ref[...]
    el_ref[...] = ee[:, 0:1]
    er_ref[...] = ee[:, 1:2]


def _tc_next(p, sp, w, a):
    return pl.pallas_call(
        _tc_next_body,
        grid=(1,),
        in_specs=[
            pl.BlockSpec((NC, N, D), lambda i: (0, 0, 0)),
            pl.BlockSpec((NW, 1, N), lambda i: (0, 0, 0)),
            pl.BlockSpec((D, D), lambda i: (0, 0)),
            pl.BlockSpec((D, 2), lambda i: (0, 0)),
        ],
        out_specs=[
            pl.BlockSpec((N, D), lambda i: (0, 0)),
            pl.BlockSpec((N, D), lambda i: (0, 0)),
            pl.BlockSpec((N, 1), lambda i: (0, 0)),
            pl.BlockSpec((N, 1), lambda i: (0, 0)),
        ],
        out_shape=[
            jax.ShapeDtypeStruct((N, D), jnp.float32),
            jax.ShapeDtypeStruct((N, D), jnp.float32),
            jax.ShapeDtypeStruct((N, 1), jnp.float32),
            jax.ShapeDtypeStruct((N, 1), jnp.float32),
        ],
    )(p, sp, w, a)


def _tc_mlp_body(p_ref, sp_ref, h0_ref, h1_ref, w0a_ref, w0b_ref, w0c_ref,
                 b0_ref, w1_ref, b1_ref, w2_ref, b2_ref, out_ref):
    g = _norm_elu(p_ref, sp_ref)
    t = (h0_ref[...] @ w0a_ref[...] + h1_ref[...] @ w0b_ref[...]
         + g @ w0c_ref[...] + b0_ref[...])
    t = jnp.maximum(t, 0.0)
    t = jnp.maximum(t @ w1_ref[...] + b1_ref[...], 0.0)
    t = jnp.maximum(t @ w2_ref[...] + b2_ref[...], 0.0)
    out_ref[...] = t


def _tc_mlp(p, sp, h0, h1, w0a, w0b, w0c, b0, w1, b1, w2, b2):
    return pl.pallas_call(
        _tc_mlp_body,
        grid=(1,),
        in_specs=[
            pl.BlockSpec((NC, N, D), lambda i: (0, 0, 0)),
            pl.BlockSpec((NW, 1, N), lambda i: (0, 0, 0)),
            pl.BlockSpec((N, D), lambda i: (0, 0)),
            pl.BlockSpec((N, D), lambda i: (0, 0)),
            pl.BlockSpec((D, D), lambda i: (0, 0)),
            pl.BlockSpec((D, D), lambda i: (0, 0)),
            pl.BlockSpec((D, D), lambda i: (0, 0)),
            pl.BlockSpec((D,), lambda i: (0,)),
            pl.BlockSpec((D, D), lambda i: (0, 0)),
            pl.BlockSpec((D,), lambda i: (0,)),
            pl.BlockSpec((D, 1), lambda i: (0, 0)),
            pl.BlockSpec((1,), lambda i: (0,)),
        ],
        out_specs=pl.BlockSpec((N, 1), lambda i: (0, 0)),
        out_shape=jax.ShapeDtypeStruct((N, 1), jnp.float32),
    )(p, sp, h0, h1, w0a, w0b, w0c, b0, w1, b1, w2, b2)


@functools.lru_cache(maxsize=1)
def _build_sc_kernels():
  mesh = plsc.VectorSubcoreMesh(core_axis_name="c", subcore_axis_name="s")

  @functools.partial(
      pl.kernel,
      out_type=[
          jax.ShapeDtypeStruct((NW, 1, N), jnp.float32),
          jax.ShapeDtypeStruct((NC, N, D), jnp.float32),
      ],
      mesh=mesh,
      compiler_params=pltpu.CompilerParams(needs_layout_passes=False,
                                           use_tc_tiling_on_sc=False),
      scratch_types=[
          pltpu.VMEM((N,), jnp.float32),
          pltpu.VMEM((N,), jnp.float32),
          pltpu.VMEM((NCHUNK, CH), jnp.int32),
          pltpu.VMEM((NCHUNK, CH), jnp.int32),
          pltpu.VMEM((N,), jnp.float32),
          pltpu.VMEM((2, CH, D), jnp.float32),
          pltpu.VMEM((CH, D), jnp.float32),
          pltpu.VMEM((CPR, D), jnp.float32),
          pltpu.VMEM_SHARED((N, D), jnp.float32),
          pltpu.SemaphoreType.DMA,
          pltpu.SemaphoreType.DMA,
      ],
  )
  def sc_conv(el_hbm, er_hbm, src_hbm, dst_hbm, wh_hbm, sp_hbm, out_hbm,
              el_v, er_v, src_v, dst_v, s_v, rows_v, scaled_v, tmp_v,
              acc_sh, gsem0, gsem1):
    c = lax.axis_index("c")
    s = lax.axis_index("s")
    wid = s * NC + c
    last = s == NS - 1

    zz = jnp.zeros((L,), jnp.float32)

    def zbody(i, carry):
      for q in range(D // L):
        tmp_v[i, pl.ds(q * L, L)] = zz
      return carry

    lax.fori_loop(0, CPR, zbody, 0)
    for k in range(RPT0 // CPR):
      pltpu.sync_copy(tmp_v, acc_sh.at[pl.ds(s * RPT0 + k * CPR, CPR)])

    @pl.when(last)
    def _():
      pltpu.sync_copy(tmp_v.at[pl.ds(0, RPT_LAST - RPT0)],
                      acc_sh.at[pl.ds(N - (RPT_LAST - RPT0), RPT_LAST - RPT0)])

    def szero(i, carry):
      s_v[pl.ds(i * L, L)] = zz
      return carry

    lax.fori_loop(0, N // L, szero, 0)

    pltpu.sync_copy(el_hbm, el_v)
    pltpu.sync_copy(er_hbm, er_v)
    pltpu.sync_copy(src_hbm.at[wid], src_v)
    pltpu.sync_copy(dst_hbm.at[wid], dst_v)
    plsc.subcore_barrier()

    gsems = (gsem0, gsem1)

    def gstart(ci, b):
      pltpu.async_copy(wh_hbm.at[src_v.at[ci]], rows_v.at[b], gsems[b])

    def gwait(b):
      pltpu.make_async_copy(wh_hbm.at[pl.ds(0, CH)], rows_v.at[b],
                            gsems[b]).wait()

    def process(ci, b):
      exvs = []
      for g in range(CH // L):
        sl = pl.ds(g * L, L)
        sidx = src_v[ci, sl]
        didx = dst_v[ci, sl]
        ev = plsc.load_gather(el_v, [sidx]) + plsc.load_gather(er_v, [didx])
        ev = jnp.where(ev >= 0, ev, NEG * ev)
        exv = jnp.exp(ev)
        plsc.addupdate_scatter(s_v, [didx], exv)
        exvs.append(exv)
      gwait(b)
      for g in range(CH // L):
        for j in range(L):
          e = g * L + j
          wvec = lax.gather(
              exvs[g], jnp.full((L, 1), j, jnp.int32),
              lax.GatherDimensionNumbers(offset_dims=(),
                                         collapsed_slice_dims=(0,),
                                         start_index_map=(0,)),
              (1,), mode=lax.GatherScatterMode.PROMISE_IN_BOUNDS)
          for q in range(D // L):
            ql = pl.ds(q * L, L)
            scaled_v[e, ql] = rows_v[b, e, ql] * wvec
      pltpu.sync_copy(scaled_v, acc_sh.at[dst_v.at[ci]], add=True)

    gstart(0, 0)

    def cbody(k, carry):
      ci0 = 2 * k
      gstart(ci0 + 1, 1)
      process(ci0, 0)
      gstart(ci0 + 2, 0)
      process(ci0 + 1, 1)
      return carry

    lax.fori_loop(0, (NCHUNK - 1) // 2, cbody, 0)
    process(NCHUNK - 1, 0)

    pltpu.sync_copy(s_v, sp_hbm.at[wid, 0])
    plsc.subcore_barrier()

    for k in range(RPT0 // CPR):
      rows = pl.ds(s * RPT0 + k * CPR, CPR)
      pltpu.sync_copy(acc_sh.at[rows], tmp_v)
      pltpu.sync_copy(tmp_v, out_hbm.at[c, rows])

    @pl.when(last)
    def _():
      tail = RPT_LAST - RPT0
      rows = pl.ds(N - tail, tail)
      pltpu.sync_copy(acc_sh.at[rows], tmp_v.at[pl.ds(0, tail)])
      pltpu.sync_copy(tmp_v.at[pl.ds(0, tail)], out_hbm.at[c, rows])

  return sc_conv


def _conv(wh, el, er, src3, dst3):
    sc_conv = _build_sc_kernels()
    return sc_conv(el.reshape(N), er.reshape(N), src3, dst3, wh)


def kernel(inputs, edge_index, W_emb, W0, al0, ar0, W1, al1, ar1, W2, al2, ar2,
           O0w, O0b, O1w, O1b, O2w, O2b):
    src3 = edge_index[0].reshape(NW, NCHUNK, CH)
    dst3 = edge_index[1].reshape(NW, NCHUNK, CH)
    a0 = jnp.stack([al0.reshape(D), ar0.reshape(D)], axis=1)
    a1 = jnp.stack([al1.reshape(D), ar1.reshape(D)], axis=1)
    a2 = jnp.stack([al2.reshape(D), ar2.reshape(D)], axis=1)

    wh0, el0, er0 = _tc_embed(inputs, W_emb, W0, a0)
    sp0, p0 = _conv(wh0, el0, er0, src3, dst3)
    h0, wh1, el1, er1 = _tc_next(p0, sp0, W1, a1)
    sp1, p1 = _conv(wh1, el1, er1, src3, dst3)
    h1, wh2, el2, er2 = _tc_next(p1, sp1, W2, a2)
    sp2, p2 = _conv(wh2, el2, er2, src3, dst3)
    return _tc_mlp(p2, sp2, h0, h1, O0w[0:D], O0w[D:2 * D], O0w[2 * D:3 * D],
                   O0b, O1w, O1b, O2w, O2b)


# speedup vs baseline: 66.4614x; 1.0975x over previous
import functools

import jax
import jax.numpy as jnp
from jax import lax
from jax.experimental import pallas as pl
from jax.experimental.pallas import tpu as pltpu
from jax.experimental.pallas import tpu_sc as plsc

N = 10000
E = 320000
IN_DIM = 128
D = 64
NEG = 0.2

NC = 2
NS = 16
L = 16
NW = NC * NS
EPW = E // NW
CH = 80
NCHUNK = EPW // CH
RPT0 = 624
RPT_LAST = N - (NS - 1) * RPT0
CPR = 208
RB = 2000


def _tc_embed_body(x_ref, wemb_ref, w_ref, a_ref, wh_ref, el_ref, er_ref):
    h = jnp.tanh(x_ref[...] @ wemb_ref[...])
    wh = h @ w_ref[...]
    wh_ref[...] = wh
    ee = wh @ a_ref[...]
    el_ref[...] = ee[:, 0:1]
    er_ref[...] = ee[:, 1:2]


def _tc_embed(x, wemb, w, a):
    return pl.pallas_call(
        _tc_embed_body,
        grid=(N // RB,),
        in_specs=[
            pl.BlockSpec((RB, IN_DIM), lambda i: (i, 0)),
            pl.BlockSpec((IN_DIM, D), lambda i: (0, 0)),
            pl.BlockSpec((D, D), lambda i: (0, 0)),
            pl.BlockSpec((D, 2), lambda i: (0, 0)),
        ],
        out_specs=[
            pl.BlockSpec((RB, D), lambda i: (i, 0)),
            pl.BlockSpec((RB, 1), lambda i: (i, 0)),
            pl.BlockSpec((RB, 1), lambda i: (i, 0)),
        ],
        out_shape=[
            jax.ShapeDtypeStruct((N, D), jnp.float32),
            jax.ShapeDtypeStruct((N, 1), jnp.float32),
            jax.ShapeDtypeStruct((N, 1), jnp.float32),
        ],
    )(x, wemb, w, a)


def _norm_elu(p_ref, sp_ref):
    s = jnp.sum(sp_ref[...], axis=(0, 1))
    x = (p_ref[0] + p_ref[1]) * (1.0 / (s + 1e-9))[:, None]
    return jnp.where(x > 0, x, jnp.exp(x) - 1.0)


def _tc_next_body(p_ref, sp_ref, w_ref, a_ref, g_ref, wh_ref, el_ref, er_ref):
    g = _norm_elu(p_ref, sp_ref)
    g_ref[...] = g
    wh = g @ w_ref[...]
    wh_ref[...] = wh
    ee = wh @ a_ref[...]
    el_ref[...] = ee[:, 0:1]
    er_ref[...] = ee[:, 1:2]


def _tc_next(p, sp, w, a):
    return pl.pallas_call(
        _tc_next_body,
        grid=(1,),
        in_specs=[
            pl.BlockSpec((NC, N, D), lambda i: (0, 0, 0)),
            pl.BlockSpec((NW, 1, N), lambda i: (0, 0, 0)),
            pl.BlockSpec((D, D), lambda i: (0, 0)),
            pl.BlockSpec((D, 2), lambda i: (0, 0)),
        ],
        out_specs=[
            pl.BlockSpec((N, D), lambda i: (0, 0)),
            pl.BlockSpec((N, D), lambda i: (0, 0)),
            pl.BlockSpec((N, 1), lambda i: (0, 0)),
            pl.BlockSpec((N, 1), lambda i: (0, 0)),
        ],
        out_shape=[
            jax.ShapeDtypeStruct((N, D), jnp.float32),
            jax.ShapeDtypeStruct((N, D), jnp.float32),
            jax.ShapeDtypeStruct((N, 1), jnp.float32),
            jax.ShapeDtypeStruct((N, 1), jnp.float32),
        ],
    )(p, sp, w, a)


def _tc_mlp_body(p_ref, sp_ref, h0_ref, h1_ref, w0a_ref, w0b_ref, w0c_ref,
                 b0_ref, w1_ref, b1_ref, w2_ref, b2_ref, out_ref):
    g = _norm_elu(p_ref, sp_ref)
    t = (h0_ref[...] @ w0a_ref[...] + h1_ref[...] @ w0b_ref[...]
         + g @ w0c_ref[...] + b0_ref[...])
    t = jnp.maximum(t, 0.0)
    t = jnp.maximum(t @ w1_ref[...] + b1_ref[...], 0.0)
    t = jnp.maximum(t @ w2_ref[...] + b2_ref[...], 0.0)
    out_ref[...] = t


def _tc_mlp(p, sp, h0, h1, w0a, w0b, w0c, b0, w1, b1, w2, b2):
    return pl.pallas_call(
        _tc_mlp_body,
        grid=(1,),
        in_specs=[
            pl.BlockSpec((NC, N, D), lambda i: (0, 0, 0)),
            pl.BlockSpec((NW, 1, N), lambda i: (0, 0, 0)),
            pl.BlockSpec((N, D), lambda i: (0, 0)),
            pl.BlockSpec((N, D), lambda i: (0, 0)),
            pl.BlockSpec((D, D), lambda i: (0, 0)),
            pl.BlockSpec((D, D), lambda i: (0, 0)),
            pl.BlockSpec((D, D), lambda i: (0, 0)),
            pl.BlockSpec((D,), lambda i: (0,)),
            pl.BlockSpec((D, D), lambda i: (0, 0)),
            pl.BlockSpec((D,), lambda i: (0,)),
            pl.BlockSpec((D, 1), lambda i: (0, 0)),
            pl.BlockSpec((1,), lambda i: (0,)),
        ],
        out_specs=pl.BlockSpec((N, 1), lambda i: (0, 0)),
        out_shape=jax.ShapeDtypeStruct((N, 1), jnp.float32),
    )(p, sp, h0, h1, w0a, w0b, w0c, b0, w1, b1, w2, b2)


@functools.lru_cache(maxsize=1)
def _build_sc_kernels():
  mesh = plsc.VectorSubcoreMesh(core_axis_name="c", subcore_axis_name="s")

  @functools.partial(
      pl.kernel,
      out_type=[
          jax.ShapeDtypeStruct((NW, 1, N), jnp.float32),
          jax.ShapeDtypeStruct((NC, N, D), jnp.float32),
      ],
      mesh=mesh,
      compiler_params=pltpu.CompilerParams(needs_layout_passes=False,
                                           use_tc_tiling_on_sc=False),
      scratch_types=[
          pltpu.VMEM((N,), jnp.float32),
          pltpu.VMEM((N,), jnp.float32),
          pltpu.VMEM((NCHUNK, CH), jnp.int32),
          pltpu.VMEM((NCHUNK, CH), jnp.int32),
          pltpu.VMEM((N,), jnp.float32),
          pltpu.VMEM((2, CH, D), jnp.float32),
          pltpu.VMEM((2, CH, D), jnp.float32),
          pltpu.VMEM((CPR, D), jnp.float32),
          pltpu.VMEM_SHARED((N, D), jnp.float32),
          pltpu.SemaphoreType.DMA,
          pltpu.SemaphoreType.DMA,
          pltpu.SemaphoreType.DMA,
          pltpu.SemaphoreType.DMA,
      ],
  )
  def sc_conv(el_hbm, er_hbm, src_hbm, dst_hbm, wh_hbm, sp_hbm, out_hbm,
              el_v, er_v, src_v, dst_v, s_v, rows_v, scaled_v, tmp_v,
              acc_sh, gsem0, gsem1, ssem0, ssem1):
    c = lax.axis_index("c")
    s = lax.axis_index("s")
    wid = s * NC + c
    last = s == NS - 1

    zz = jnp.zeros((L,), jnp.float32)

    def zbody(i, carry):
      for q in range(D // L):
        tmp_v[i, pl.ds(q * L, L)] = zz
      return carry

    lax.fori_loop(0, CPR, zbody, 0)
    for k in range(RPT0 // CPR):
      pltpu.sync_copy(tmp_v, acc_sh.at[pl.ds(s * RPT0 + k * CPR, CPR)])

    @pl.when(last)
    def _():
      pltpu.sync_copy(tmp_v.at[pl.ds(0, RPT_LAST - RPT0)],
                      acc_sh.at[pl.ds(N - (RPT_LAST - RPT0), RPT_LAST - RPT0)])

    def szero(i, carry):
      s_v[pl.ds(i * L, L)] = zz
      return carry

    lax.fori_loop(0, N // L, szero, 0)

    pltpu.sync_copy(el_hbm, el_v)
    pltpu.sync_copy(er_hbm, er_v)
    pltpu.sync_copy(src_hbm.at[wid], src_v)
    pltpu.sync_copy(dst_hbm.at[wid], dst_v)
    plsc.subcore_barrier()

    gsems = (gsem0, gsem1)
    ssems = (ssem0, ssem1)

    def gstart(ci, b):
      pltpu.async_copy(wh_hbm.at[src_v.at[ci]], rows_v.at[b], gsems[b])

    def gwait(b):
      pltpu.make_async_copy(wh_hbm.at[pl.ds(0, CH)], rows_v.at[b],
                            gsems[b]).wait()

    def swait(b):
      pltpu.make_async_copy(scaled_v.at[b], acc_sh.at[dst_v.at[0]],
                            ssems[b]).wait()

    def process(ci, b):
      exvs = []
      for g in range(CH // L):
        sl = pl.ds(g * L, L)
        sidx = src_v[ci, sl]
        didx = dst_v[ci, sl]
        ev = plsc.load_gather(el_v, [sidx]) + plsc.load_gather(er_v, [didx])
        ev = jnp.where(ev >= 0, ev, NEG * ev)
        exv = jnp.exp(ev)
        plsc.addupdate_scatter(s_v, [didx], exv)
        exvs.append(exv)
      gwait(b)
      swait(b)
      for g in range(CH // L):
        for j in range(L):
          e = g * L + j
          wvec = lax.gather(
              exvs[g], jnp.full((L, 1), j, jnp.int32),
              lax.GatherDimensionNumbers(offset_dims=(),
                                         collapsed_slice_dims=(0,),
                                         start_index_map=(0,)),
              (1,), mode=lax.GatherScatterMode.PROMISE_IN_BOUNDS)
          for q in range(D // L):
            ql = pl.ds(q * L, L)
            scaled_v[b, e, ql] = rows_v[b, e, ql] * wvec
      pltpu.async_copy(scaled_v.at[b], acc_sh.at[dst_v.at[ci]], ssems[b],
                       add=True)

    def sbzero(i, carry):
      for q in range(D // L):
        scaled_v[0, i, pl.ds(q * L, L)] = zz
        scaled_v[1, i, pl.ds(q * L, L)] = zz
      return carry

    lax.fori_loop(0, CH, sbzero, 0)
    pltpu.async_copy(scaled_v.at[0], acc_sh.at[dst_v.at[0]], ssem0, add=True)
    pltpu.async_copy(scaled_v.at[1], acc_sh.at[dst_v.at[0]], ssem1, add=True)
    gstart(0, 0)

    def cbody(k, carry):
      ci0 = 2 * k
      gstart(ci0 + 1, 1)
      process(ci0, 0)
      gstart(ci0 + 2, 0)
      process(ci0 + 1, 1)
      return carry

    lax.fori_loop(0, (NCHUNK - 1) // 2, cbody, 0)
    process(NCHUNK - 1, 0)
    swait(0)
    swait(1)

    pltpu.sync_copy(s_v, sp_hbm.at[wid, 0])
    plsc.subcore_barrier()

    for k in range(RPT0 // CPR):
      rows = pl.ds(s * RPT0 + k * CPR, CPR)
      pltpu.sync_copy(acc_sh.at[rows], tmp_v)
      pltpu.sync_copy(tmp_v, out_hbm.at[c, rows])

    @pl.when(last)
    def _():
      tail = RPT_LAST - RPT0
      rows = pl.ds(N - tail, tail)
      pltpu.sync_copy(acc_sh.at[rows], tmp_v.at[pl.ds(0, tail)])
      pltpu.sync_copy(tmp_v.at[pl.ds(0, tail)], out_hbm.at[c, rows])

  return sc_conv


def _conv(wh, el, er, src3, dst3):
    sc_conv = _build_sc_kernels()
    return sc_conv(el.reshape(N), er.reshape(N), src3, dst3, wh)


def kernel(inputs, edge_index, W_emb, W0, al0, ar0, W1, al1, ar1, W2, al2, ar2,
           O0w, O0b, O1w, O1b, O2w, O2b):
    src3 = edge_index[0].reshape(NW, NCHUNK, CH)
    dst3 = edge_index[1].reshape(NW, NCHUNK, CH)
    a0 = jnp.stack([al0.reshape(D), ar0.reshape(D)], axis=1)
    a1 = jnp.stack([al1.reshape(D), ar1.reshape(D)], axis=1)
    a2 = jnp.stack([al2.reshape(D), ar2.reshape(D)], axis=1)

    wh0, el0, er0 = _tc_embed(inputs, W_emb, W0, a0)
    sp0, p0 = _conv(wh0, el0, er0, src3, dst3)
    h0, wh1, el1, er1 = _tc_next(p0, sp0, W1, a1)
    sp1, p1 = _conv(wh1, el1, er1, src3, dst3)
    h1, wh2, el2, er2 = _tc_next(p1, sp1, W2, a2)
    sp2, p2 = _conv(wh2, el2, er2, src3, dst3)
    return _tc_mlp(p2, sp2, h0, h1, O0w[0:D], O0w[D:2 * D], O0w[2 * D:3 * D],
                   O0b, O1w, O1b, O2w, O2b)


# speedup vs baseline: 73.4764x; 1.1055x over previous
import functools

import jax
import jax.numpy as jnp
from jax import lax
from jax.experimental import pallas as pl
from jax.experimental.pallas import tpu as pltpu
from jax.experimental.pallas import tpu_sc as plsc

N = 10000
E = 320000
IN_DIM = 128
D = 64
NEG = 0.2

NC = 2
NS = 16
L = 16
NW = NC * NS
EPW = E // NW
CH = 80
NCHUNK = EPW // CH
RPT0 = 624
RPT_LAST = N - (NS - 1) * RPT0
CPR = 104
RB = 2000


def _tc_embed_body(x_ref, wemb_ref, w_ref, a_ref, wh_ref, el_ref, er_ref):
    h = jnp.tanh(x_ref[...] @ wemb_ref[...])
    wh = h @ w_ref[...]
    wh_ref[...] = wh
    ee = wh @ a_ref[...]
    el_ref[...] = ee[:, 0:1]
    er_ref[...] = ee[:, 1:2]


def _tc_embed(x, wemb, w, a):
    return pl.pallas_call(
        _tc_embed_body,
        grid=(N // RB,),
        in_specs=[
            pl.BlockSpec((RB, IN_DIM), lambda i: (i, 0)),
            pl.BlockSpec((IN_DIM, D), lambda i: (0, 0)),
            pl.BlockSpec((D, D), lambda i: (0, 0)),
            pl.BlockSpec((D, 2), lambda i: (0, 0)),
        ],
        out_specs=[
            pl.BlockSpec((RB, D), lambda i: (i, 0)),
            pl.BlockSpec((RB, 1), lambda i: (i, 0)),
            pl.BlockSpec((RB, 1), lambda i: (i, 0)),
        ],
        out_shape=[
            jax.ShapeDtypeStruct((N, D), jnp.float32),
            jax.ShapeDtypeStruct((N, 1), jnp.float32),
            jax.ShapeDtypeStruct((N, 1), jnp.float32),
        ],
    )(x, wemb, w, a)


def _norm_elu(p_ref, sp_ref):
    s = jnp.sum(sp_ref[...], axis=(0, 1))
    x = (p_ref[0] + p_ref[1]) * (1.0 / (s + 1e-9))[:, None]
    return jnp.where(x > 0, x, jnp.exp(x) - 1.0)


def _tc_next_body(p_ref, sp_ref, w_ref, a_ref, g_ref, wh_ref, el_ref, er_ref):
    g = _norm_elu(p_ref, sp_ref)
    g_ref[...] = g
    wh = g @ w_ref[...]
    wh_ref[...] = wh
    ee = wh @ a_ref[...]
    el_ref[...] = ee[:, 0:1]
    er_ref[...] = ee[:, 1:2]


def _tc_next(p, sp, w, a):
    return pl.pallas_call(
        _tc_next_body,
        grid=(1,),
        in_specs=[
            pl.BlockSpec((NC, N, D), lambda i: (0, 0, 0)),
            pl.BlockSpec((NW, 1, N), lambda i: (0, 0, 0)),
            pl.BlockSpec((D, D), lambda i: (0, 0)),
            pl.BlockSpec((D, 2), lambda i: (0, 0)),
        ],
        out_specs=[
            pl.BlockSpec((N, D), lambda i: (0, 0)),
            pl.BlockSpec((N, D), lambda i: (0, 0)),
            pl.BlockSpec((N, 1), lambda i: (0, 0)),
            pl.BlockSpec((N, 1), lambda i: (0, 0)),
        ],
        out_shape=[
            jax.ShapeDtypeStruct((N, D), jnp.float32),
            jax.ShapeDtypeStruct((N, D), jnp.float32),
            jax.ShapeDtypeStruct((N, 1), jnp.float32),
            jax.ShapeDtypeStruct((N, 1), jnp.float32),
        ],
    )(p, sp, w, a)


def _tc_mlp_body(p_ref, sp_ref, h0_ref, h1_ref, w0a_ref, w0b_ref, w0c_ref,
                 b0_ref, w1_ref, b1_ref, w2_ref, b2_ref, out_ref):
    g = _norm_elu(p_ref, sp_ref)
    t = (h0_ref[...] @ w0a_ref[...] + h1_ref[...] @ w0b_ref[...]
         + g @ w0c_ref[...] + b0_ref[...])
    t = jnp.maximum(t, 0.0)
    t = jnp.maximum(t @ w1_ref[...] + b1_ref[...], 0.0)
    t = jnp.maximum(t @ w2_ref[...] + b2_ref[...], 0.0)
    out_ref[...] = t


def _tc_mlp(p, sp, h0, h1, w0a, w0b, w0c, b0, w1, b1, w2, b2):
    return pl.pallas_call(
        _tc_mlp_body,
        grid=(1,),
        in_specs=[
            pl.BlockSpec((NC, N, D), lambda i: (0, 0, 0)),
            pl.BlockSpec((NW, 1, N), lambda i: (0, 0, 0)),
            pl.BlockSpec((N, D), lambda i: (0, 0)),
            pl.BlockSpec((N, D), lambda i: (0, 0)),
            pl.BlockSpec((D, D), lambda i: (0, 0)),
            pl.BlockSpec((D, D), lambda i: (0, 0)),
            pl.BlockSpec((D, D), lambda i: (0, 0)),
            pl.BlockSpec((D,), lambda i: (0,)),
            pl.BlockSpec((D, D), lambda i: (0, 0)),
            pl.BlockSpec((D,), lambda i: (0,)),
            pl.BlockSpec((D, 1), lambda i: (0, 0)),
            pl.BlockSpec((1,), lambda i: (0,)),
        ],
        out_specs=pl.BlockSpec((N, 1), lambda i: (0, 0)),
        out_shape=jax.ShapeDtypeStruct((N, 1), jnp.float32),
    )(p, sp, h0, h1, w0a, w0b, w0c, b0, w1, b1, w2, b2)


@functools.lru_cache(maxsize=1)
def _build_sc_kernels():
  mesh = plsc.VectorSubcoreMesh(core_axis_name="c", subcore_axis_name="s")

  @functools.partial(
      pl.kernel,
      out_type=[
          jax.ShapeDtypeStruct((NW, 1, N), jnp.float32),
          jax.ShapeDtypeStruct((NC, N, D), jnp.float32),
      ],
      mesh=mesh,
      compiler_params=pltpu.CompilerParams(needs_layout_passes=False,
                                           use_tc_tiling_on_sc=False),
      scratch_types=[
          pltpu.VMEM((N,), jnp.float32),
          pltpu.VMEM((N,), jnp.float32),
          pltpu.VMEM((NCHUNK, CH), jnp.int32),
          pltpu.VMEM((NCHUNK, CH), jnp.int32),
          pltpu.VMEM((N,), jnp.float32),
          pltpu.VMEM((3, CH, D), jnp.float32),
          pltpu.VMEM((3, CH, D), jnp.float32),
          pltpu.VMEM((CPR, D), jnp.float32),
          pltpu.VMEM_SHARED((N, D), jnp.float32),
          pltpu.SemaphoreType.DMA,
          pltpu.SemaphoreType.DMA,
          pltpu.SemaphoreType.DMA,
          pltpu.SemaphoreType.DMA,
          pltpu.SemaphoreType.DMA,
          pltpu.SemaphoreType.DMA,
      ],
  )
  def sc_conv(el_hbm, er_hbm, src_hbm, dst_hbm, wh_hbm, sp_hbm, out_hbm,
              el_v, er_v, src_v, dst_v, s_v, rows_v, scaled_v, tmp_v,
              acc_sh, gsem0, gsem1, gsem2, ssem0, ssem1, ssem2):
    c = lax.axis_index("c")
    s = lax.axis_index("s")
    wid = s * NC + c
    last = s == NS - 1

    zz = jnp.zeros((L,), jnp.float32)

    def zbody(i, carry):
      for q in range(D // L):
        tmp_v[i, pl.ds(q * L, L)] = zz
      return carry

    lax.fori_loop(0, CPR, zbody, 0)
    for k in range(RPT0 // CPR):
      pltpu.sync_copy(tmp_v, acc_sh.at[pl.ds(s * RPT0 + k * CPR, CPR)])

    @pl.when(last)
    def _():
      pltpu.sync_copy(tmp_v.at[pl.ds(0, RPT_LAST - RPT0)],
                      acc_sh.at[pl.ds(N - (RPT_LAST - RPT0), RPT_LAST - RPT0)])

    def szero(i, carry):
      s_v[pl.ds(i * L, L)] = zz
      return carry

    lax.fori_loop(0, N // L, szero, 0)

    pltpu.sync_copy(el_hbm, el_v)
    pltpu.sync_copy(er_hbm, er_v)
    pltpu.sync_copy(src_hbm.at[wid], src_v)
    pltpu.sync_copy(dst_hbm.at[wid], dst_v)
    plsc.subcore_barrier()

    gsems = (gsem0, gsem1, gsem2)
    ssems = (ssem0, ssem1, ssem2)

    def gstart(ci, b):
      pltpu.async_copy(wh_hbm.at[src_v.at[ci]], rows_v.at[b], gsems[b])

    def gwait(b):
      pltpu.make_async_copy(wh_hbm.at[pl.ds(0, CH)], rows_v.at[b],
                            gsems[b]).wait()

    def swait(b):
      pltpu.make_async_copy(scaled_v.at[b], acc_sh.at[dst_v.at[0]],
                            ssems[b]).wait()

    def process(ci, b):
      exvs = []
      for g in range(CH // L):
        sl = pl.ds(g * L, L)
        sidx = src_v[ci, sl]
        didx = dst_v[ci, sl]
        ev = plsc.load_gather(el_v, [sidx]) + plsc.load_gather(er_v, [didx])
        ev = jnp.where(ev >= 0, ev, NEG * ev)
        exv = jnp.exp(ev)
        plsc.addupdate_scatter(s_v, [didx], exv)
        exvs.append(exv)
      gwait(b)
      swait(b)
      for g in range(CH // L):
        for j in range(L):
          e = g * L + j
          wvec = lax.gather(
              exvs[g], jnp.full((L, 1), j, jnp.int32),
              lax.GatherDimensionNumbers(offset_dims=(),
                                         collapsed_slice_dims=(0,),
                                         start_index_map=(0,)),
              (1,), mode=lax.GatherScatterMode.PROMISE_IN_BOUNDS)
          for q in range(D // L):
            ql = pl.ds(q * L, L)
            scaled_v[b, e, ql] = rows_v[b, e, ql] * wvec
      pltpu.async_copy(scaled_v.at[b], acc_sh.at[dst_v.at[ci]], ssems[b],
                       add=True)

    def sbzero(i, carry):
      for q in range(D // L):
        scaled_v[0, i, pl.ds(q * L, L)] = zz
        scaled_v[1, i, pl.ds(q * L, L)] = zz
        scaled_v[2, i, pl.ds(q * L, L)] = zz
      return carry

    lax.fori_loop(0, CH, sbzero, 0)
    pltpu.async_copy(scaled_v.at[0], acc_sh.at[dst_v.at[0]], ssem0, add=True)
    pltpu.async_copy(scaled_v.at[1], acc_sh.at[dst_v.at[0]], ssem1, add=True)
    pltpu.async_copy(scaled_v.at[2], acc_sh.at[dst_v.at[0]], ssem2, add=True)
    gstart(0, 0)
    gstart(1, 1)

    def cbody(k, carry):
      ci0 = 3 * k
      gstart(ci0 + 2, 2)
      process(ci0, 0)
      gstart(ci0 + 3, 0)
      process(ci0 + 1, 1)
      gstart(ci0 + 4, 1)
      process(ci0 + 2, 2)
      return carry

    lax.fori_loop(0, (NCHUNK - 2) // 3, cbody, 0)
    process(NCHUNK - 2, 0)
    process(NCHUNK - 1, 1)
    swait(0)
    swait(1)
    swait(2)

    pltpu.sync_copy(s_v, sp_hbm.at[wid, 0])
    plsc.subcore_barrier()

    for k in range(RPT0 // CPR):
      rows = pl.ds(s * RPT0 + k * CPR, CPR)
      pltpu.sync_copy(acc_sh.at[rows], tmp_v)
      pltpu.sync_copy(tmp_v, out_hbm.at[c, rows])

    @pl.when(last)
    def _():
      tail = RPT_LAST - RPT0
      rows = pl.ds(N - tail, tail)
      pltpu.sync_copy(acc_sh.at[rows], tmp_v.at[pl.ds(0, tail)])
      pltpu.sync_copy(tmp_v.at[pl.ds(0, tail)], out_hbm.at[c, rows])

  return sc_conv


def _conv(wh, el, er, src3, dst3):
    sc_conv = _build_sc_kernels()
    return sc_conv(el.reshape(N), er.reshape(N), src3, dst3, wh)


def kernel(inputs, edge_index, W_emb, W0, al0, ar0, W1, al1, ar1, W2, al2, ar2,
           O0w, O0b, O1w, O1b, O2w, O2b):
    src3 = edge_index[0].reshape(NW, NCHUNK, CH)
    dst3 = edge_index[1].reshape(NW, NCHUNK, CH)
    a0 = jnp.stack([al0.reshape(D), ar0.reshape(D)], axis=1)
    a1 = jnp.stack([al1.reshape(D), ar1.reshape(D)], axis=1)
    a2 = jnp.stack([al2.reshape(D), ar2.reshape(D)], axis=1)

    wh0, el0, er0 = _tc_embed(inputs, W_emb, W0, a0)
    sp0, p0 = _conv(wh0, el0, er0, src3, dst3)
    h0, wh1, el1, er1 = _tc_next(p0, sp0, W1, a1)
    sp1, p1 = _conv(wh1, el1, er1, src3, dst3)
    h1, wh2, el2, er2 = _tc_next(p1, sp1, W2, a2)
    sp2, p2 = _conv(wh2, el2, er2, src3, dst3)
    return _tc_mlp(p2, sp2, h0, h1, O0w[0:D], O0w[D:2 * D], O0w[2 * D:3 * D],
                   O0b, O1w, O1b, O2w, O2b)


# speedup vs baseline: 79.4414x; 1.0812x over previous
import functools

import jax
import jax.numpy as jnp
from jax import lax
from jax.experimental import pallas as pl
from jax.experimental.pallas import tpu as pltpu
from jax.experimental.pallas import tpu_sc as plsc

N = 10000
E = 320000
IN_DIM = 128
D = 64
NEG = 0.2

NC = 2
NS = 16
L = 16
NW = NC * NS
EPW = E // NW
CH = 80
NCHUNK = EPW // CH
RPT0 = 624
RPT_LAST = N - (NS - 1) * RPT0
CPR = 104
RB = 2000


def _tc_embed_body(x_ref, wemb_ref, w_ref, a_ref, wh_ref, elr_ref):
    h = jnp.tanh(x_ref[...] @ wemb_ref[...])
    wh = h @ w_ref[...]
    wh_ref[...] = wh
    elr_ref[...] = wh @ a_ref[...]


def _tc_embed(x, wemb, w, a):
    return pl.pallas_call(
        _tc_embed_body,
        grid=(N // RB,),
        in_specs=[
            pl.BlockSpec((RB, IN_DIM), lambda i: (i, 0)),
            pl.BlockSpec((IN_DIM, D), lambda i: (0, 0)),
            pl.BlockSpec((D, D), lambda i: (0, 0)),
            pl.BlockSpec((D, 2), lambda i: (0, 0)),
        ],
        out_specs=[
            pl.BlockSpec((RB, D), lambda i: (i, 0)),
            pl.BlockSpec((RB, 2), lambda i: (i, 0)),
        ],
        out_shape=[
            jax.ShapeDtypeStruct((N, D), jnp.float32),
            jax.ShapeDtypeStruct((N, 2), jnp.float32),
        ],
    )(x, wemb, w, a)


def _norm_elu(p_ref, sp_ref):
    s = jnp.sum(sp_ref[...], axis=(0, 1))
    x = (p_ref[0] + p_ref[1]) * (1.0 / (s + 1e-9))[:, None]
    return jnp.where(x > 0, x, jnp.exp(x) - 1.0)


def _tc_next_body(p_ref, sp_ref, w_ref, a_ref, g_ref, wh_ref, elr_ref):
    g = _norm_elu(p_ref, sp_ref)
    g_ref[...] = g
    wh = g @ w_ref[...]
    wh_ref[...] = wh
    elr_ref[...] = wh @ a_ref[...]


def _tc_next(p, sp, w, a):
    return pl.pallas_call(
        _tc_next_body,
        grid=(1,),
        in_specs=[
            pl.BlockSpec((NC, N, D), lambda i: (0, 0, 0)),
            pl.BlockSpec((NW, 1, N), lambda i: (0, 0, 0)),
            pl.BlockSpec((D, D), lambda i: (0, 0)),
            pl.BlockSpec((D, 2), lambda i: (0, 0)),
        ],
        out_specs=[
            pl.BlockSpec((N, D), lambda i: (0, 0)),
            pl.BlockSpec((N, D), lambda i: (0, 0)),
            pl.BlockSpec((N, 2), lambda i: (0, 0)),
        ],
        out_shape=[
            jax.ShapeDtypeStruct((N, D), jnp.float32),
            jax.ShapeDtypeStruct((N, D), jnp.float32),
            jax.ShapeDtypeStruct((N, 2), jnp.float32),
        ],
    )(p, sp, w, a)


def _tc_mlp_body(p_ref, sp_ref, h0_ref, h1_ref, w0a_ref, w0b_ref, w0c_ref,
                 b0_ref, w1_ref, b1_ref, w2_ref, b2_ref, out_ref):
    g = _norm_elu(p_ref, sp_ref)
    t = (h0_ref[...] @ w0a_ref[...] + h1_ref[...] @ w0b_ref[...]
         + g @ w0c_ref[...] + b0_ref[...])
    t = jnp.maximum(t, 0.0)
    t = jnp.maximum(t @ w1_ref[...] + b1_ref[...], 0.0)
    t = jnp.maximum(t @ w2_ref[...] + b2_ref[...], 0.0)
    out_ref[...] = t


def _tc_mlp(p, sp, h0, h1, w0a, w0b, w0c, b0, w1, b1, w2, b2):
    return pl.pallas_call(
        _tc_mlp_body,
        grid=(1,),
        in_specs=[
            pl.BlockSpec((NC, N, D), lambda i: (0, 0, 0)),
            pl.BlockSpec((NW, 1, N), lambda i: (0, 0, 0)),
            pl.BlockSpec((N, D), lambda i: (0, 0)),
            pl.BlockSpec((N, D), lambda i: (0, 0)),
            pl.BlockSpec((D, D), lambda i: (0, 0)),
            pl.BlockSpec((D, D), lambda i: (0, 0)),
            pl.BlockSpec((D, D), lambda i: (0, 0)),
            pl.BlockSpec((D,), lambda i: (0,)),
            pl.BlockSpec((D, D), lambda i: (0, 0)),
            pl.BlockSpec((D,), lambda i: (0,)),
            pl.BlockSpec((D, 1), lambda i: (0, 0)),
            pl.BlockSpec((1,), lambda i: (0,)),
        ],
        out_specs=pl.BlockSpec((N, 1), lambda i: (0, 0)),
        out_shape=jax.ShapeDtypeStruct((N, 1), jnp.float32),
    )(p, sp, h0, h1, w0a, w0b, w0c, b0, w1, b1, w2, b2)


@functools.lru_cache(maxsize=1)
def _build_sc_kernels():
  mesh = plsc.VectorSubcoreMesh(core_axis_name="c", subcore_axis_name="s")

  @functools.partial(
      pl.kernel,
      out_type=[
          jax.ShapeDtypeStruct((NW, 1, N), jnp.float32),
          jax.ShapeDtypeStruct((NC, N, D), jnp.float32),
      ],
      mesh=mesh,
      compiler_params=pltpu.CompilerParams(needs_layout_passes=False,
                                           use_tc_tiling_on_sc=False),
      scratch_types=[
          pltpu.VMEM((2 * N,), jnp.float32),
          pltpu.VMEM((NCHUNK, CH), jnp.int32),
          pltpu.VMEM((NCHUNK, CH), jnp.int32),
          pltpu.VMEM((N,), jnp.float32),
          pltpu.VMEM((3, CH, D), jnp.float32),
          pltpu.VMEM((3, CH, D), jnp.float32),
          pltpu.VMEM((CPR, D), jnp.float32),
          pltpu.VMEM_SHARED((N, D), jnp.float32),
          pltpu.SemaphoreType.DMA,
          pltpu.SemaphoreType.DMA,
          pltpu.SemaphoreType.DMA,
          pltpu.SemaphoreType.DMA,
          pltpu.SemaphoreType.DMA,
          pltpu.SemaphoreType.DMA,
      ],
  )
  def sc_conv(elr_hbm, edge_hbm, wh_hbm, sp_hbm, out_hbm,
              elr_v, src_v, dst_v, s_v, rows_v, scaled_v, tmp_v,
              acc_sh, gsem0, gsem1, gsem2, ssem0, ssem1, ssem2):
    c = lax.axis_index("c")
    s = lax.axis_index("s")
    wid = s * NC + c
    last = s == NS - 1

    zz = jnp.zeros((L,), jnp.float32)

    def zbody(i, carry):
      for q in range(D // L):
        tmp_v[i, pl.ds(q * L, L)] = zz
      return carry

    lax.fori_loop(0, CPR, zbody, 0)
    for k in range(RPT0 // CPR):
      pltpu.sync_copy(tmp_v, acc_sh.at[pl.ds(s * RPT0 + k * CPR, CPR)])

    @pl.when(last)
    def _():
      pltpu.sync_copy(tmp_v.at[pl.ds(0, RPT_LAST - RPT0)],
                      acc_sh.at[pl.ds(N - (RPT_LAST - RPT0), RPT_LAST - RPT0)])

    def szero(i, carry):
      s_v[pl.ds(i * L, L)] = zz
      return carry

    lax.fori_loop(0, N // L, szero, 0)

    pltpu.sync_copy(elr_hbm, elr_v)
    pltpu.sync_copy(edge_hbm.at[0, wid], src_v)
    pltpu.sync_copy(edge_hbm.at[1, wid], dst_v)
    plsc.subcore_barrier()

    gsems = (gsem0, gsem1, gsem2)
    ssems = (ssem0, ssem1, ssem2)

    def gstart(ci, b):
      pltpu.async_copy(wh_hbm.at[src_v.at[ci]], rows_v.at[b], gsems[b])

    def gwait(b):
      pltpu.make_async_copy(wh_hbm.at[pl.ds(0, CH)], rows_v.at[b],
                            gsems[b]).wait()

    def swait(b):
      pltpu.make_async_copy(scaled_v.at[b], acc_sh.at[dst_v.at[0]],
                            ssems[b]).wait()

    def process(ci, b):
      exvs = []
      for g in range(CH // L):
        sl = pl.ds(g * L, L)
        sidx = src_v[ci, sl]
        didx = dst_v[ci, sl]
        ev = (plsc.load_gather(elr_v, [sidx * 2])
              + plsc.load_gather(elr_v, [didx * 2 + 1]))
        ev = jnp.where(ev >= 0, ev, NEG * ev)
        exv = jnp.exp(ev)
        plsc.addupdate_scatter(s_v, [didx], exv)
        exvs.append(exv)
      gwait(b)
      swait(b)
      for g in range(CH // L):
        for j in range(L):
          e = g * L + j
          wvec = lax.gather(
              exvs[g], jnp.full((L, 1), j, jnp.int32),
              lax.GatherDimensionNumbers(offset_dims=(),
                                         collapsed_slice_dims=(0,),
                                         start_index_map=(0,)),
              (1,), mode=lax.GatherScatterMode.PROMISE_IN_BOUNDS)
          for q in range(D // L):
            ql = pl.ds(q * L, L)
            scaled_v[b, e, ql] = rows_v[b, e, ql] * wvec
      pltpu.async_copy(scaled_v.at[b], acc_sh.at[dst_v.at[ci]], ssems[b],
                       add=True)

    def sbzero(i, carry):
      for q in range(D // L):
        scaled_v[0, i, pl.ds(q * L, L)] = zz
        scaled_v[1, i, pl.ds(q * L, L)] = zz
        scaled_v[2, i, pl.ds(q * L, L)] = zz
      return carry

    lax.fori_loop(0, CH, sbzero, 0)
    pltpu.async_copy(scaled_v.at[0], acc_sh.at[dst_v.at[0]], ssem0, add=True)
    pltpu.async_copy(scaled_v.at[1], acc_sh.at[dst_v.at[0]], ssem1, add=True)
    pltpu.async_copy(scaled_v.at[2], acc_sh.at[dst_v.at[0]], ssem2, add=True)
    gstart(0, 0)
    gstart(1, 1)

    def cbody(k, carry):
      ci0 = 3 * k
      gstart(ci0 + 2, 2)
      process(ci0, 0)
      gstart(ci0 + 3, 0)
      process(ci0 + 1, 1)
      gstart(ci0 + 4, 1)
      process(ci0 + 2, 2)
      return carry

    lax.fori_loop(0, (NCHUNK - 2) // 3, cbody, 0)
    process(NCHUNK - 2, 0)
    process(NCHUNK - 1, 1)
    swait(0)
    swait(1)
    swait(2)

    pltpu.sync_copy(s_v, sp_hbm.at[wid, 0])
    plsc.subcore_barrier()

    for k in range(RPT0 // CPR):
      rows = pl.ds(s * RPT0 + k * CPR, CPR)
      pltpu.sync_copy(acc_sh.at[rows], tmp_v)
      pltpu.sync_copy(tmp_v, out_hbm.at[c, rows])

    @pl.when(last)
    def _():
      tail = RPT_LAST - RPT0
      rows = pl.ds(N - tail, tail)
      pltpu.sync_copy(acc_sh.at[rows], tmp_v.at[pl.ds(0, tail)])
      pltpu.sync_copy(tmp_v.at[pl.ds(0, tail)], out_hbm.at[c, rows])

  return sc_conv


def _conv(wh, elr, edge4):
    sc_conv = _build_sc_kernels()
    return sc_conv(elr.reshape(2 * N), edge4, wh)


def kernel(inputs, edge_index, W_emb, W0, al0, ar0, W1, al1, ar1, W2, al2, ar2,
           O0w, O0b, O1w, O1b, O2w, O2b):
    edge4 = edge_index.reshape(2, NW, NCHUNK, CH)
    a0 = jnp.stack([al0.reshape(D), ar0.reshape(D)], axis=1)
    a1 = jnp.stack([al1.reshape(D), ar1.reshape(D)], axis=1)
    a2 = jnp.stack([al2.reshape(D), ar2.reshape(D)], axis=1)

    wh0, elr0 = _tc_embed(inputs, W_emb, W0, a0)
    sp0, p0 = _conv(wh0, elr0, edge4)
    h0, wh1, elr1 = _tc_next(p0, sp0, W1, a1)
    sp1, p1 = _conv(wh1, elr1, edge4)
    h1, wh2, elr2 = _tc_next(p1, sp1, W2, a2)
    sp2, p2 = _conv(wh2, elr2, edge4)
    return _tc_mlp(p2, sp2, h0, h1, O0w[0:D], O0w[D:2 * D], O0w[2 * D:3 * D],
                   O0b, O1w, O1b, O2w, O2b)
